# Initial kernel scaffold; baseline (speedup 1.0000x reference)
import jax
import jax.numpy as jnp
from jax.experimental import pallas as pl


def kernel(x, edge_index, batch_index, edge_attr, W1, a1_src, a1_dst, b1, gamma, beta, W2, a2_src, a2_dst, b2, Wout, bout):
    raise NotImplementedError("write your pallas kernel here")


# speedup vs baseline: 69.2000x; 69.2000x over previous
import functools

import jax
import jax.numpy as jnp
from jax import lax
from jax.experimental import pallas as pl
from jax.experimental.pallas import tpu as pltpu
from jax.experimental.pallas import tpu_sc as plsc

N = 100000
E = 6400000
D = 8
NUM_GRAPHS = 256

NP = 100096
SLICE = NP // 16
NW = 32
EW = E // NW
K = 1600
CHUNKS = EW // K
VK = K // 16
_PIECES = [(0, K), (K, K), (2 * K, K), (3 * K, SLICE - 3 * K)]

_mesh = plsc.VectorSubcoreMesh(core_axis_name="c", subcore_axis_name="s")


def _edge_body(src_h, dst_h, as_h, ad_h, ht_h, c_h, nums_h, dens_h,
               asp, adp, hp0, hp1, hp2, hp3, hp4, hp5, hp6, hp7,
               np0, np1, np2, np3, np4, np5, np6, np7, dnp,
               sidx, didx, av, bv, wv, hv, cvm, sem):
    hps = (hp0, hp1, hp2, hp3, hp4, hp5, hp6, hp7)
    nps = (np0, np1, np2, np3, np4, np5, np6, np7)
    c = lax.axis_index("c")
    s = lax.axis_index("s")
    wid = c * 16 + s
    off = s * SLICE

    for (p, sz) in _PIECES:
        pltpu.sync_copy(as_h.at[pl.ds(off + p, sz)], hv.at[pl.ds(0, sz)])
        pltpu.sync_copy(hv.at[pl.ds(0, sz)], asp.at[pl.ds(off + p, sz)])
        pltpu.sync_copy(ad_h.at[pl.ds(off + p, sz)], hv.at[pl.ds(0, sz)])
        pltpu.sync_copy(hv.at[pl.ds(0, sz)], adp.at[pl.ds(off + p, sz)])
    for j in range(D):
        for (p, sz) in _PIECES:
            pltpu.sync_copy(ht_h.at[pl.ds(j * NP + off + p, sz)],
                            hv.at[pl.ds(0, sz)])
            pltpu.sync_copy(hv.at[pl.ds(0, sz)], hps[j].at[pl.ds(off + p, sz)])

    def _zb_body(i, carry):
        wv[pl.ds(i * 16, 16)] = jnp.zeros((16,), jnp.float32)
        return carry
    lax.fori_loop(0, VK, _zb_body, 0)
    for (p, sz) in _PIECES:
        pltpu.sync_copy(wv.at[pl.ds(0, sz)], dnp.at[pl.ds(off + p, sz)])
        for j in range(D):
            pltpu.sync_copy(wv.at[pl.ds(0, sz)],
                            nps[j].at[pl.ds(off + p, sz)])

    pltpu.sync_copy(c_h, cvm)
    plsc.subcore_barrier()

    cv = cvm[...]

    def _chunk(i, carry):
        base = wid * EW + i * K
        pltpu.sync_copy(src_h.at[pl.ds(base, K)], sidx)
        pltpu.sync_copy(dst_h.at[pl.ds(base, K)], didx)
        pltpu.async_copy(asp.at[sidx], av, sem).wait()
        pltpu.async_copy(adp.at[didx], bv, sem).wait()

        def _w_body(t, carry2):
            a = av[pl.ds(t * 16, 16)] + bv[pl.ds(t * 16, 16)]
            a = jnp.where(a > 0, a, 0.2 * a)
            wv[pl.ds(t * 16, 16)] = jnp.exp(a - cv)
            return carry2
        lax.fori_loop(0, VK, _w_body, 0)
        pltpu.sync_copy(wv, dnp.at[didx], add=True)

        for j in range(D):
            pltpu.async_copy(hps[j].at[sidx], hv, sem).wait()

            def _m_body(t, carry2):
                hv[pl.ds(t * 16, 16)] = (hv[pl.ds(t * 16, 16)] *
                                         wv[pl.ds(t * 16, 16)])
                return carry2
            lax.fori_loop(0, VK, _m_body, 0)
            pltpu.sync_copy(hv, nps[j].at[didx], add=True)
        return carry
    lax.fori_loop(0, CHUNKS, _chunk, 0)

    plsc.subcore_barrier()

    for (p, sz) in _PIECES:
        pltpu.sync_copy(dnp.at[pl.ds(off + p, sz)], hv.at[pl.ds(0, sz)])
        pltpu.sync_copy(hv.at[pl.ds(0, sz)],
                        dens_h.at[pl.ds(c * NP + off + p, sz)])
        for j in range(D):
            pltpu.sync_copy(nps[j].at[pl.ds(off + p, sz)],
                            hv.at[pl.ds(0, sz)])
            pltpu.sync_copy(hv.at[pl.ds(0, sz)],
                            nums_h.at[pl.ds(c * (D * NP) + j * NP + off + p,
                                            sz)])


_edge_pass = functools.partial(
    pl.kernel,
    out_type=(jax.ShapeDtypeStruct((2 * D * NP,), jnp.float32),
              jax.ShapeDtypeStruct((2 * NP,), jnp.float32)),
    mesh=_mesh,
    scratch_types=(
        [pltpu.VMEM_SHARED((NP,), jnp.float32)] * 2
        + [pltpu.VMEM_SHARED((NP,), jnp.float32)] * D
        + [pltpu.VMEM_SHARED((NP,), jnp.float32)] * D
        + [pltpu.VMEM_SHARED((NP,), jnp.float32)]
        + [pltpu.VMEM((K,), jnp.int32)] * 2
        + [pltpu.VMEM((K,), jnp.float32)] * 4
        + [pltpu.VMEM((16,), jnp.float32)]
        + [pltpu.SemaphoreType.DMA]
    ),
)(_edge_body)


def _leaky(x):
    return jnp.where(x > 0, x, 0.2 * x)


def _gat_layer(x, src, dst, W, a_src, a_dst, bias):
    h = x @ W
    as_ = h @ a_src
    ad_ = h @ a_dst
    C = _leaky(jnp.max(as_) + jnp.max(ad_))
    pad = NP - N
    asp = jnp.pad(as_, (0, pad))
    adp = jnp.pad(ad_, (0, pad))
    htp = jnp.pad(h.T, ((0, 0), (0, pad))).reshape(-1)
    cvec = jnp.full((16,), C, jnp.float32)
    nums_f, dens_f = _edge_pass(src, dst, asp, adp, htp, cvec)
    nums = nums_f.reshape(2, D, NP)[:, :, :N]
    dens = dens_f.reshape(2, NP)[:, :N]
    wself = jnp.exp(_leaky(as_ + ad_) - C)
    den = dens[0] + dens[1] + wself
    num = nums[0] + nums[1] + wself[None, :] * h.T
    return (num / den[None, :]).T + bias


def kernel(x, edge_index, batch_index, edge_attr, W1, a1_src, a1_dst, b1,
           gamma, beta, W2, a2_src, a2_dst, b2, Wout, bout):
    src = edge_index[0]
    dst = edge_index[1]
    h = _gat_layer(x, src, dst, W1, a1_src, a1_dst, b1)
    h = jnp.tanh(h)
    mean = jnp.mean(h, axis=0)
    var = jnp.var(h, axis=0)
    h = (h - mean) / jnp.sqrt(var + 1e-5) * gamma + beta
    h = _gat_layer(h, src, dst, W2, a2_src, a2_dst, b2)
    h = jnp.tanh(h)
    gmax = jax.ops.segment_max(h, batch_index, num_segments=NUM_GRAPHS)
    gsum = jax.ops.segment_sum(h, batch_index, num_segments=NUM_GRAPHS)
    counts = jax.ops.segment_sum(jnp.ones((N, 1), dtype=h.dtype),
                                 batch_index, num_segments=NUM_GRAPHS)
    gmean = gsum / jnp.maximum(counts, 1.0)
    pooled = jnp.concatenate([gmax, gmean], axis=1)
    return pooled @ Wout + bout


# speedup vs baseline: 87.6561x; 1.2667x over previous
import functools

import jax
import jax.numpy as jnp
from jax import lax
from jax.experimental import pallas as pl
from jax.experimental.pallas import tpu as pltpu
from jax.experimental.pallas import tpu_sc as plsc

N = 100000
E = 6400000
D = 8
NUM_GRAPHS = 256

NP = 100096
SLICE = NP // 16
NW = 32
EW = E // NW
K = 800
CHUNKS = EW // K
VK = K // 16
_PIECES = [(i * K, K) for i in range(SLICE // K)] + [
    ((SLICE // K) * K, SLICE % K)]

_mesh = plsc.VectorSubcoreMesh(core_axis_name="c", subcore_axis_name="s")


def _edge_body(src_h, dst_h, as_h, ad_h, ht_h, c_h, nums_h, dens_h,
               asp, adp, hp0, hp1, hp2, hp3, hp4, hp5, hp6, hp7,
               np0, np1, np2, np3, np4, np5, np6, np7, dnp,
               sidx, didx, av, bv, wv, hb0, hb1, hb2, cvm,
               semi, sema, sg0, sg1, sg2, ss0, ss1, ss2, sden):
    hps = (hp0, hp1, hp2, hp3, hp4, hp5, hp6, hp7)
    nps = (np0, np1, np2, np3, np4, np5, np6, np7)
    hbufs = (hb0, hb1, hb2)
    gsems = (sg0, sg1, sg2)
    ssems = (ss0, ss1, ss2)
    c = lax.axis_index("c")
    s = lax.axis_index("s")
    wid = c * 16 + s
    off = s * SLICE

    for (p, sz) in _PIECES:
        pltpu.sync_copy(as_h.at[pl.ds(off + p, sz)], hb0.at[pl.ds(0, sz)])
        pltpu.sync_copy(hb0.at[pl.ds(0, sz)], asp.at[pl.ds(off + p, sz)])
        pltpu.sync_copy(ad_h.at[pl.ds(off + p, sz)], hb0.at[pl.ds(0, sz)])
        pltpu.sync_copy(hb0.at[pl.ds(0, sz)], adp.at[pl.ds(off + p, sz)])
    for j in range(D):
        for (p, sz) in _PIECES:
            pltpu.sync_copy(ht_h.at[pl.ds(j * NP + off + p, sz)],
                            hb0.at[pl.ds(0, sz)])
            pltpu.sync_copy(hb0.at[pl.ds(0, sz)],
                            hps[j].at[pl.ds(off + p, sz)])

    def _zb_body(i, carry):
        wv[pl.ds(i * 16, 16)] = jnp.zeros((16,), jnp.float32)
        return carry
    lax.fori_loop(0, VK, _zb_body, 0)
    for (p, sz) in _PIECES:
        pltpu.sync_copy(wv.at[pl.ds(0, sz)], dnp.at[pl.ds(off + p, sz)])
        for j in range(D):
            pltpu.sync_copy(wv.at[pl.ds(0, sz)],
                            nps[j].at[pl.ds(off + p, sz)])

    pltpu.sync_copy(c_h, cvm)
    plsc.subcore_barrier()

    cv = cvm[...]

    def _chunk(i, carry):
        base = wid * EW + i * K
        c1 = pltpu.async_copy(src_h.at[pl.ds(base, K)], sidx, semi)
        c2 = pltpu.async_copy(dst_h.at[pl.ds(base, K)], didx, semi)
        c1.wait()
        c2.wait()
        ga = pltpu.async_copy(asp.at[sidx], av, sema)
        gb = pltpu.async_copy(adp.at[didx], bv, sema)
        g = [None, None, None]
        g[0] = pltpu.async_copy(hp0.at[sidx], hb0, sg0)
        ga.wait()
        gb.wait()

        def _w_body(t, carry2):
            a = av[pl.ds(t * 16, 16)] + bv[pl.ds(t * 16, 16)]
            a = jnp.where(a > 0, a, 0.2 * a)
            wv[pl.ds(t * 16, 16)] = jnp.exp(a - cv)
            return carry2
        lax.fori_loop(0, VK, _w_body, 0, unroll=5)
        pltpu.sync_copy(wv, dnp.at[didx], add=True)

        for j in range(D):
            if j < D - 1:
                nb = (j + 1) % 3
                g[nb] = pltpu.async_copy(hps[j + 1].at[sidx],
                                         hbufs[nb], gsems[nb])
            b = j % 3
            g[b].wait()
            hv = hbufs[b]

            def _m_body(t, carry2):
                hv[pl.ds(t * 16, 16)] = (hv[pl.ds(t * 16, 16)] *
                                         wv[pl.ds(t * 16, 16)])
                return carry2
            lax.fori_loop(0, VK, _m_body, 0, unroll=5)
            pltpu.sync_copy(hv, nps[j].at[didx], add=True)
        return carry
    lax.fori_loop(0, CHUNKS, _chunk, 0)

    plsc.subcore_barrier()

    for (p, sz) in _PIECES:
        pltpu.sync_copy(dnp.at[pl.ds(off + p, sz)], hb0.at[pl.ds(0, sz)])
        pltpu.sync_copy(hb0.at[pl.ds(0, sz)],
                        dens_h.at[pl.ds(c * NP + off + p, sz)])
        for j in range(D):
            pltpu.sync_copy(nps[j].at[pl.ds(off + p, sz)],
                            hb0.at[pl.ds(0, sz)])
            pltpu.sync_copy(hb0.at[pl.ds(0, sz)],
                            nums_h.at[pl.ds(c * (D * NP) + j * NP + off + p,
                                            sz)])


_edge_pass = functools.partial(
    pl.kernel,
    out_type=(jax.ShapeDtypeStruct((2 * D * NP,), jnp.float32),
              jax.ShapeDtypeStruct((2 * NP,), jnp.float32)),
    mesh=_mesh,
    scratch_types=(
        [pltpu.VMEM_SHARED((NP,), jnp.float32)] * 2
        + [pltpu.VMEM_SHARED((NP,), jnp.float32)] * D
        + [pltpu.VMEM_SHARED((NP,), jnp.float32)] * D
        + [pltpu.VMEM_SHARED((NP,), jnp.float32)]
        + [pltpu.VMEM((K,), jnp.int32)] * 2
        + [pltpu.VMEM((K,), jnp.float32)] * 3
        + [pltpu.VMEM((K,), jnp.float32)] * 3
        + [pltpu.VMEM((16,), jnp.float32)]
        + [pltpu.SemaphoreType.DMA] * 9
    ),
)(_edge_body)


def _leaky(x):
    return jnp.where(x > 0, x, 0.2 * x)


def _gat_layer(x, src, dst, W, a_src, a_dst, bias):
    h = x @ W
    as_ = h @ a_src
    ad_ = h @ a_dst
    C = _leaky(jnp.max(as_) + jnp.max(ad_))
    pad = NP - N
    asp = jnp.pad(as_, (0, pad))
    adp = jnp.pad(ad_, (0, pad))
    htp = jnp.pad(h.T, ((0, 0), (0, pad))).reshape(-1)
    cvec = jnp.full((16,), C, jnp.float32)
    nums_f, dens_f = _edge_pass(src, dst, asp, adp, htp, cvec)
    nums = nums_f.reshape(2, D, NP)[:, :, :N]
    dens = dens_f.reshape(2, NP)[:, :N]
    wself = jnp.exp(_leaky(as_ + ad_) - C)
    den = dens[0] + dens[1] + wself
    num = nums[0] + nums[1] + wself[None, :] * h.T
    return (num / den[None, :]).T + bias


def kernel(x, edge_index, batch_index, edge_attr, W1, a1_src, a1_dst, b1,
           gamma, beta, W2, a2_src, a2_dst, b2, Wout, bout):
    src = edge_index[0]
    dst = edge_index[1]
    h = _gat_layer(x, src, dst, W1, a1_src, a1_dst, b1)
    h = jnp.tanh(h)
    mean = jnp.mean(h, axis=0)
    var = jnp.var(h, axis=0)
    h = (h - mean) / jnp.sqrt(var + 1e-5) * gamma + beta
    h = _gat_layer(h, src, dst, W2, a2_src, a2_dst, b2)
    h = jnp.tanh(h)
    gmax = jax.ops.segment_max(h, batch_index, num_segments=NUM_GRAPHS)
    gsum = jax.ops.segment_sum(h, batch_index, num_segments=NUM_GRAPHS)
    counts = jax.ops.segment_sum(jnp.ones((N, 1), dtype=h.dtype),
                                 batch_index, num_segments=NUM_GRAPHS)
    gmean = gsum / jnp.maximum(counts, 1.0)
    pooled = jnp.concatenate([gmax, gmean], axis=1)
    return pooled @ Wout + bout


# speedup vs baseline: 93.9511x; 1.0718x over previous
import functools

import jax
import jax.numpy as jnp
from jax import lax
from jax.experimental import pallas as pl
from jax.experimental.pallas import tpu as pltpu
from jax.experimental.pallas import tpu_sc as plsc

N = 100000
E = 6400000
D = 8
NUM_GRAPHS = 256

NP = 100096
SLICE = NP // 16
NW = 32
EW = E // NW
K = 800
CHUNKS = EW // K
VK = K // 16
_PIECES = [(i * K, K) for i in range(SLICE // K)] + [
    ((SLICE // K) * K, SLICE % K)]

_mesh = plsc.VectorSubcoreMesh(core_axis_name="c", subcore_axis_name="s")


def _edge_body(src_h, dst_h, as_h, ad_h, ht_h, c_h, nums_h, dens_h,
               asp, adp, hp0, hp1, hp2, hp3, hp4, hp5, hp6, hp7,
               np0, np1, np2, np3, np4, np5, np6, np7, dnp,
               sidx, didx, av, bv, wv, hb0, hb1, hb2, cvm,
               semi, sema, sg0, sg1, sg2, ss0, ss1, ss2, sden):
    hps = (hp0, hp1, hp2, hp3, hp4, hp5, hp6, hp7)
    nps = (np0, np1, np2, np3, np4, np5, np6, np7)
    hbufs = (hb0, hb1, hb2)
    gsems = (sg0, sg1, sg2)
    ssems = (ss0, ss1, ss2)
    c = lax.axis_index("c")
    s = lax.axis_index("s")
    wid = c * 16 + s
    off = s * SLICE

    for (p, sz) in _PIECES:
        pltpu.sync_copy(as_h.at[pl.ds(off + p, sz)], hb0.at[pl.ds(0, sz)])
        pltpu.sync_copy(hb0.at[pl.ds(0, sz)], asp.at[pl.ds(off + p, sz)])
        pltpu.sync_copy(ad_h.at[pl.ds(off + p, sz)], hb0.at[pl.ds(0, sz)])
        pltpu.sync_copy(hb0.at[pl.ds(0, sz)], adp.at[pl.ds(off + p, sz)])
    for j in range(D):
        for (p, sz) in _PIECES:
            pltpu.sync_copy(ht_h.at[pl.ds(j * NP + off + p, sz)],
                            hb0.at[pl.ds(0, sz)])
            pltpu.sync_copy(hb0.at[pl.ds(0, sz)],
                            hps[j].at[pl.ds(off + p, sz)])

    def _zb_body(i, carry):
        wv[pl.ds(i * 16, 16)] = jnp.zeros((16,), jnp.float32)
        return carry
    lax.fori_loop(0, VK, _zb_body, 0)
    for (p, sz) in _PIECES:
        pltpu.sync_copy(wv.at[pl.ds(0, sz)], dnp.at[pl.ds(off + p, sz)])
        for j in range(D):
            pltpu.sync_copy(wv.at[pl.ds(0, sz)],
                            nps[j].at[pl.ds(off + p, sz)])

    pltpu.sync_copy(c_h, cvm)
    plsc.subcore_barrier()

    cv = cvm[...]

    def _chunk(i, carry):
        base = wid * EW + i * K
        c1 = pltpu.async_copy(src_h.at[pl.ds(base, K)], sidx, semi)
        c2 = pltpu.async_copy(dst_h.at[pl.ds(base, K)], didx, semi)
        c1.wait()
        c2.wait()
        ga = pltpu.async_copy(asp.at[sidx], av, sema)
        gb = pltpu.async_copy(adp.at[didx], bv, sema)
        g = [None, None, None]
        g[0] = pltpu.async_copy(hp0.at[sidx], hb0, sg0)
        ga.wait()
        gb.wait()

        def _w_body(t, carry2):
            a = av[pl.ds(t * 16, 16)] + bv[pl.ds(t * 16, 16)]
            a = jnp.where(a > 0, a, 0.2 * a)
            wv[pl.ds(t * 16, 16)] = jnp.exp(a - cv)
            return carry2
        lax.fori_loop(0, VK, _w_body, 0, unroll=5)
        pltpu.sync_copy(wv, dnp.at[didx], add=True)

        for j in range(D):
            if j < D - 1:
                nb = (j + 1) % 3
                g[nb] = pltpu.async_copy(hps[j + 1].at[sidx],
                                         hbufs[nb], gsems[nb])
            b = j % 3
            g[b].wait()
            hv = hbufs[b]

            def _m_body(t, carry2):
                hv[pl.ds(t * 16, 16)] = (hv[pl.ds(t * 16, 16)] *
                                         wv[pl.ds(t * 16, 16)])
                return carry2
            lax.fori_loop(0, VK, _m_body, 0, unroll=5)
            pltpu.sync_copy(hv, nps[j].at[didx], add=True)
        return carry
    lax.fori_loop(0, CHUNKS, _chunk, 0)

    plsc.subcore_barrier()

    for (p, sz) in _PIECES:
        pltpu.sync_copy(dnp.at[pl.ds(off + p, sz)], hb0.at[pl.ds(0, sz)])
        pltpu.sync_copy(hb0.at[pl.ds(0, sz)],
                        dens_h.at[pl.ds(c * NP + off + p, sz)])
        for j in range(D):
            pltpu.sync_copy(nps[j].at[pl.ds(off + p, sz)],
                            hb0.at[pl.ds(0, sz)])
            pltpu.sync_copy(hb0.at[pl.ds(0, sz)],
                            nums_h.at[pl.ds(c * (D * NP) + j * NP + off + p,
                                            sz)])


_edge_pass = functools.partial(
    pl.kernel,
    out_type=(jax.ShapeDtypeStruct((2 * D * NP,), jnp.float32),
              jax.ShapeDtypeStruct((2 * NP,), jnp.float32)),
    mesh=_mesh,
    scratch_types=(
        [pltpu.VMEM_SHARED((NP,), jnp.float32)] * 2
        + [pltpu.VMEM_SHARED((NP,), jnp.float32)] * D
        + [pltpu.VMEM_SHARED((NP,), jnp.float32)] * D
        + [pltpu.VMEM_SHARED((NP,), jnp.float32)]
        + [pltpu.VMEM((K,), jnp.int32)] * 2
        + [pltpu.VMEM((K,), jnp.float32)] * 3
        + [pltpu.VMEM((K,), jnp.float32)] * 3
        + [pltpu.VMEM((16,), jnp.float32)]
        + [pltpu.SemaphoreType.DMA] * 9
    ),
)(_edge_body)


BP = 2048
NB = 49
NPAD = NB * BP
OUT_DIM = 2145 * 2


def _pool_body(bt_ref, ht_ref, wout_ref, bout_ref, out_ref,
               gmax_sc, gsum_sc, cnt_sc):
    pid = pl.program_id(0)
    bt = bt_ref[0, 0, :]
    gids = lax.broadcasted_iota(jnp.int32, (1, NUM_GRAPHS), 1)
    onehot = bt[:, None] == gids
    oh_f = onehot.astype(jnp.float32)
    hT = jnp.tanh(ht_ref[...])
    gsum_part = lax.dot_general(oh_f, hT,
                                (((0,), (1,)), ((), ())))
    cnt_part = jnp.sum(oh_f, axis=0)[:, None]
    cols = []
    for ci in range(D):
        m = jnp.where(onehot, hT[ci, :][:, None], -jnp.inf)
        cols.append(jnp.max(m, axis=0))
    gmax_part = jnp.stack(cols, axis=1)

    @pl.when(pid == 0)
    def _init():
        gmax_sc[...] = gmax_part
        gsum_sc[...] = gsum_part
        cnt_sc[...] = cnt_part

    @pl.when(pid > 0)
    def _acc():
        gmax_sc[...] = jnp.maximum(gmax_sc[...], gmax_part)
        gsum_sc[...] = gsum_sc[...] + gsum_part
        cnt_sc[...] = cnt_sc[...] + cnt_part

    @pl.when(pid == NB - 1)
    def _fin():
        gmean = gsum_sc[...] / jnp.maximum(cnt_sc[...], 1.0)
        pooled = jnp.concatenate([gmax_sc[...], gmean], axis=1)
        out_ref[...] = pooled @ wout_ref[...] + bout_ref[...]


def _pool_out(batch3, ht_pad, Wout, bout2):
    return pl.pallas_call(
        _pool_body,
        grid=(NB,),
        in_specs=[
            pl.BlockSpec((1, 1, BP), lambda i: (i, 0, 0)),
            pl.BlockSpec((D, BP), lambda i: (0, i)),
            pl.BlockSpec((2 * D, OUT_DIM), lambda i: (0, 0)),
            pl.BlockSpec((1, OUT_DIM), lambda i: (0, 0)),
        ],
        out_specs=pl.BlockSpec((NUM_GRAPHS, OUT_DIM), lambda i: (0, 0)),
        out_shape=jax.ShapeDtypeStruct((NUM_GRAPHS, OUT_DIM), jnp.float32),
        scratch_shapes=[
            pltpu.VMEM((NUM_GRAPHS, D), jnp.float32),
            pltpu.VMEM((NUM_GRAPHS, D), jnp.float32),
            pltpu.VMEM((NUM_GRAPHS, 1), jnp.float32),
        ],
    )(batch3, ht_pad, Wout, bout2)


def _leaky(x):
    return jnp.where(x > 0, x, 0.2 * x)


def _gat_layer(x, src, dst, W, a_src, a_dst, bias):
    h = x @ W
    as_ = h @ a_src
    ad_ = h @ a_dst
    C = _leaky(jnp.max(as_) + jnp.max(ad_))
    pad = NP - N
    asp = jnp.pad(as_, (0, pad))
    adp = jnp.pad(ad_, (0, pad))
    htp = jnp.pad(h.T, ((0, 0), (0, pad))).reshape(-1)
    cvec = jnp.full((16,), C, jnp.float32)
    nums_f, dens_f = _edge_pass(src, dst, asp, adp, htp, cvec)
    nums = nums_f.reshape(2, D, NP)[:, :, :N]
    dens = dens_f.reshape(2, NP)[:, :N]
    wself = jnp.exp(_leaky(as_ + ad_) - C)
    den = dens[0] + dens[1] + wself
    num = nums[0] + nums[1] + wself[None, :] * h.T
    return (num / den[None, :]).T + bias


def kernel(x, edge_index, batch_index, edge_attr, W1, a1_src, a1_dst, b1,
           gamma, beta, W2, a2_src, a2_dst, b2, Wout, bout):
    src = edge_index[0]
    dst = edge_index[1]
    h = _gat_layer(x, src, dst, W1, a1_src, a1_dst, b1)
    h = jnp.tanh(h)
    mean = jnp.mean(h, axis=0)
    var = jnp.var(h, axis=0)
    h = (h - mean) / jnp.sqrt(var + 1e-5) * gamma + beta
    h = _gat_layer(h, src, dst, W2, a2_src, a2_dst, b2)
    batch3 = jnp.pad(batch_index, (0, NPAD - N),
                     constant_values=-1).reshape(NB, 1, BP)
    ht_pad = jnp.pad(h.T, ((0, 0), (0, NPAD - N)))
    return _pool_out(batch3, ht_pad, Wout, bout[None, :])


# speedup vs baseline: 95.6133x; 1.0177x over previous
import functools

import jax
import jax.numpy as jnp
from jax import lax
from jax.experimental import pallas as pl
from jax.experimental.pallas import tpu as pltpu
from jax.experimental.pallas import tpu_sc as plsc

N = 100000
E = 6400000
D = 8
NUM_GRAPHS = 256

NP = 100096
SLICE = NP // 16
NW = 32
EW = E // NW
K = 800
CHUNKS = EW // K
VK = K // 16
_PIECES = [(i * K, K) for i in range(SLICE // K)] + [
    ((SLICE // K) * K, SLICE % K)]

_mesh = plsc.VectorSubcoreMesh(core_axis_name="c", subcore_axis_name="s")


def _edge_body(src_h, dst_h, as_h, ad_h, ht_h, c_h, nums_h, dens_h,
               asp, adp, hp0, hp1, hp2, hp3, hp4, hp5, hp6, hp7,
               np0, np1, np2, np3, np4, np5, np6, np7, dnp,
               sidx, didx, sidx2, didx2, av, bv, wv, hb0, hb1, hb2, cvm,
               semi, semi2, sema, sg0, sg1, sg2, ss0, ss1, ss2, sden):
    hps = (hp0, hp1, hp2, hp3, hp4, hp5, hp6, hp7)
    nps = (np0, np1, np2, np3, np4, np5, np6, np7)
    hbufs = (hb0, hb1, hb2)
    gsems = (sg0, sg1, sg2)
    ssems = (ss0, ss1, ss2)
    c = lax.axis_index("c")
    s = lax.axis_index("s")
    wid = c * 16 + s
    off = s * SLICE

    for (p, sz) in _PIECES:
        pltpu.sync_copy(as_h.at[pl.ds(off + p, sz)], hb0.at[pl.ds(0, sz)])
        pltpu.sync_copy(hb0.at[pl.ds(0, sz)], asp.at[pl.ds(off + p, sz)])
        pltpu.sync_copy(ad_h.at[pl.ds(off + p, sz)], hb0.at[pl.ds(0, sz)])
        pltpu.sync_copy(hb0.at[pl.ds(0, sz)], adp.at[pl.ds(off + p, sz)])
    for j in range(D):
        for (p, sz) in _PIECES:
            pltpu.sync_copy(ht_h.at[pl.ds(j * NP + off + p, sz)],
                            hb0.at[pl.ds(0, sz)])
            pltpu.sync_copy(hb0.at[pl.ds(0, sz)],
                            hps[j].at[pl.ds(off + p, sz)])

    def _zb_body(i, carry):
        wv[pl.ds(i * 16, 16)] = jnp.zeros((16,), jnp.float32)
        return carry
    lax.fori_loop(0, VK, _zb_body, 0)
    for (p, sz) in _PIECES:
        pltpu.sync_copy(wv.at[pl.ds(0, sz)], dnp.at[pl.ds(off + p, sz)])
        for j in range(D):
            pltpu.sync_copy(wv.at[pl.ds(0, sz)],
                            nps[j].at[pl.ds(off + p, sz)])

    pltpu.sync_copy(c_h, cvm)
    plsc.subcore_barrier()

    cv = cvm[...]

    def _do_chunk(si, di):
        ga = pltpu.async_copy(asp.at[si], av, sema)
        gb = pltpu.async_copy(adp.at[di], bv, sema)
        g = [None, None, None]
        g[0] = pltpu.async_copy(hp0.at[si], hb0, sg0)
        ga.wait()
        gb.wait()

        def _w_body(t, carry2):
            a = av[pl.ds(t * 16, 16)] + bv[pl.ds(t * 16, 16)]
            a = jnp.where(a > 0, a, 0.2 * a)
            wv[pl.ds(t * 16, 16)] = jnp.exp(a - cv)
            return carry2
        lax.fori_loop(0, VK, _w_body, 0, unroll=5)
        pltpu.sync_copy(wv, dnp.at[di], add=True)

        for j in range(D):
            if j < D - 1:
                nb = (j + 1) % 3
                g[nb] = pltpu.async_copy(hps[j + 1].at[si],
                                         hbufs[nb], gsems[nb])
            b = j % 3
            g[b].wait()
            hv = hbufs[b]

            def _m_body(t, carry2):
                hv[pl.ds(t * 16, 16)] = (hv[pl.ds(t * 16, 16)] *
                                         wv[pl.ds(t * 16, 16)])
                return carry2
            lax.fori_loop(0, VK, _m_body, 0, unroll=5)
            pltpu.sync_copy(hv, nps[j].at[di], add=True)

    def _chunk(i, carry):
        base = wid * EW + 2 * i * K
        c1 = pltpu.async_copy(src_h.at[pl.ds(base, K)], sidx, semi)
        c2 = pltpu.async_copy(dst_h.at[pl.ds(base, K)], didx, semi)
        c3 = pltpu.async_copy(src_h.at[pl.ds(base + K, K)], sidx2, semi2)
        c4 = pltpu.async_copy(dst_h.at[pl.ds(base + K, K)], didx2, semi2)
        c1.wait()
        c2.wait()
        _do_chunk(sidx, didx)
        c3.wait()
        c4.wait()
        _do_chunk(sidx2, didx2)
        return carry
    lax.fori_loop(0, CHUNKS // 2, _chunk, 0)

    plsc.subcore_barrier()

    for (p, sz) in _PIECES:
        pltpu.sync_copy(dnp.at[pl.ds(off + p, sz)], hb0.at[pl.ds(0, sz)])
        pltpu.sync_copy(hb0.at[pl.ds(0, sz)],
                        dens_h.at[pl.ds(c * NP + off + p, sz)])
        for j in range(D):
            pltpu.sync_copy(nps[j].at[pl.ds(off + p, sz)],
                            hb0.at[pl.ds(0, sz)])
            pltpu.sync_copy(hb0.at[pl.ds(0, sz)],
                            nums_h.at[pl.ds(c * (D * NP) + j * NP + off + p,
                                            sz)])


_edge_pass = functools.partial(
    pl.kernel,
    out_type=(jax.ShapeDtypeStruct((2 * D * NP,), jnp.float32),
              jax.ShapeDtypeStruct((2 * NP,), jnp.float32)),
    mesh=_mesh,
    scratch_types=(
        [pltpu.VMEM_SHARED((NP,), jnp.float32)] * 2
        + [pltpu.VMEM_SHARED((NP,), jnp.float32)] * D
        + [pltpu.VMEM_SHARED((NP,), jnp.float32)] * D
        + [pltpu.VMEM_SHARED((NP,), jnp.float32)]
        + [pltpu.VMEM((K,), jnp.int32)] * 4
        + [pltpu.VMEM((K,), jnp.float32)] * 3
        + [pltpu.VMEM((K,), jnp.float32)] * 3
        + [pltpu.VMEM((16,), jnp.float32)]
        + [pltpu.SemaphoreType.DMA] * 10
    ),
)(_edge_body)


BP = 2048
NB = 49
NPAD = NB * BP
OUT_DIM = 2145 * 2


def _pool_body(bt_ref, ht_ref, wout_ref, bout_ref, out_ref,
               gmax_sc, gsum_sc, cnt_sc):
    pid = pl.program_id(0)
    bt = bt_ref[0, 0, :]
    gids = lax.broadcasted_iota(jnp.int32, (1, NUM_GRAPHS), 1)
    onehot = bt[:, None] == gids
    oh_f = onehot.astype(jnp.float32)
    hT = jnp.tanh(ht_ref[...])
    gsum_part = lax.dot_general(oh_f, hT,
                                (((0,), (1,)), ((), ())))
    cnt_part = jnp.sum(oh_f, axis=0)[:, None]
    cols = []
    for ci in range(D):
        m = jnp.where(onehot, hT[ci, :][:, None], -jnp.inf)
        cols.append(jnp.max(m, axis=0))
    gmax_part = jnp.stack(cols, axis=1)

    @pl.when(pid == 0)
    def _init():
        gmax_sc[...] = gmax_part
        gsum_sc[...] = gsum_part
        cnt_sc[...] = cnt_part

    @pl.when(pid > 0)
    def _acc():
        gmax_sc[...] = jnp.maximum(gmax_sc[...], gmax_part)
        gsum_sc[...] = gsum_sc[...] + gsum_part
        cnt_sc[...] = cnt_sc[...] + cnt_part

    @pl.when(pid == NB - 1)
    def _fin():
        gmean = gsum_sc[...] / jnp.maximum(cnt_sc[...], 1.0)
        pooled = jnp.concatenate([gmax_sc[...], gmean], axis=1)
        out_ref[...] = pooled @ wout_ref[...] + bout_ref[...]


def _pool_out(batch3, ht_pad, Wout, bout2):
    return pl.pallas_call(
        _pool_body,
        grid=(NB,),
        in_specs=[
            pl.BlockSpec((1, 1, BP), lambda i: (i, 0, 0)),
            pl.BlockSpec((D, BP), lambda i: (0, i)),
            pl.BlockSpec((2 * D, OUT_DIM), lambda i: (0, 0)),
            pl.BlockSpec((1, OUT_DIM), lambda i: (0, 0)),
        ],
        out_specs=pl.BlockSpec((NUM_GRAPHS, OUT_DIM), lambda i: (0, 0)),
        out_shape=jax.ShapeDtypeStruct((NUM_GRAPHS, OUT_DIM), jnp.float32),
        scratch_shapes=[
            pltpu.VMEM((NUM_GRAPHS, D), jnp.float32),
            pltpu.VMEM((NUM_GRAPHS, D), jnp.float32),
            pltpu.VMEM((NUM_GRAPHS, 1), jnp.float32),
        ],
    )(batch3, ht_pad, Wout, bout2)


def _leaky(x):
    return jnp.where(x > 0, x, 0.2 * x)


def _gat_layer(x, src, dst, W, a_src, a_dst, bias):
    h = x @ W
    as_ = h @ a_src
    ad_ = h @ a_dst
    C = _leaky(jnp.max(as_) + jnp.max(ad_))
    pad = NP - N
    asp = jnp.pad(as_, (0, pad))
    adp = jnp.pad(ad_, (0, pad))
    htp = jnp.pad(h.T, ((0, 0), (0, pad))).reshape(-1)
    cvec = jnp.full((16,), C, jnp.float32)
    nums_f, dens_f = _edge_pass(src, dst, asp, adp, htp, cvec)
    nums = nums_f.reshape(2, D, NP)[:, :, :N]
    dens = dens_f.reshape(2, NP)[:, :N]
    wself = jnp.exp(_leaky(as_ + ad_) - C)
    den = dens[0] + dens[1] + wself
    num = nums[0] + nums[1] + wself[None, :] * h.T
    return (num / den[None, :]).T + bias


def kernel(x, edge_index, batch_index, edge_attr, W1, a1_src, a1_dst, b1,
           gamma, beta, W2, a2_src, a2_dst, b2, Wout, bout):
    src = edge_index[0]
    dst = edge_index[1]
    h = _gat_layer(x, src, dst, W1, a1_src, a1_dst, b1)
    h = jnp.tanh(h)
    mean = jnp.mean(h, axis=0)
    var = jnp.var(h, axis=0)
    h = (h - mean) / jnp.sqrt(var + 1e-5) * gamma + beta
    h = _gat_layer(h, src, dst, W2, a2_src, a2_dst, b2)
    batch3 = jnp.pad(batch_index, (0, NPAD - N),
                     constant_values=-1).reshape(NB, 1, BP)
    ht_pad = jnp.pad(h.T, ((0, 0), (0, NPAD - N)))
    return _pool_out(batch3, ht_pad, Wout, bout[None, :])
